# Initial kernel scaffold; baseline (speedup 1.0000x reference)
import jax
import jax.numpy as jnp
from jax.experimental import pallas as pl


def kernel(ego_fut_preds, lane_preds, lane_score_preds):
    raise NotImplementedError("write your pallas kernel here")


# speedup vs baseline: 2.6702x; 2.6702x over previous
import functools
import math

import jax
import jax.numpy as jnp
from jax import lax
from jax.experimental import pallas as pl
from jax.experimental.pallas import tpu as pltpu
from jax.experimental.pallas import tpu_sc as plsc

_B = 512
_T = 6
_NPTS = 2000
_CHUNKS = _NPTS // 16
_NW = 32
_BPW = _B // _NW


def _sc_body(lx_hbm, ly_hbm, sv_hbm, eg_hbm, out_hbm, xs, ys, svec, egov, outb):
    wid = lax.axis_index("s") * 2 + lax.axis_index("c")
    iot = lax.iota(jnp.int32, 16)

    def batch_body(i, _):
        b = wid * _BPW + i
        pltpu.sync_copy(lx_hbm.at[b], xs)
        pltpu.sync_copy(ly_hbm.at[b], ys)
        pltpu.sync_copy(sv_hbm.at[b], svec)
        pltpu.sync_copy(eg_hbm.at[b], egov)

        def tbody(c, _):
            base = c * 16
            fidx = base + iot
            vidx = lax.div(fidx, jnp.int32(20))
            sc = plsc.load_gather(svec, [vidx])
            m = sc < 0.5
            xc = xs[pl.ds(base, 16)]
            yc = ys[pl.ds(base, 16)]
            xs[pl.ds(base, 16)] = jnp.where(m, 1e6, xc * 30.0 - 15.0)
            ys[pl.ds(base, 16)] = jnp.where(m, 1e6, yc * 60.0 - 30.0)
            return 0

        lax.fori_loop(0, _CHUNKS, tbody, 0)

        ev = egov[...]
        px = []
        py = []
        ax_ = ev[0] * 0.0
        ay_ = ax_
        for t in range(_T):
            ax_ = ax_ + ev[2 * t]
            ay_ = ay_ + ev[2 * t + 1]
            px.append(ax_)
            py.append(ay_)

        big = jnp.full((16,), 3e38, jnp.float32)
        zi = jnp.zeros((16,), jnp.int32)
        init = (tuple(big for _ in range(_T)), tuple(zi for _ in range(_T)))

        def cbody(c, carry):
            mins, idxs = carry
            base = c * 16
            xc = xs[pl.ds(base, 16)]
            yc = ys[pl.ds(base, 16)]
            fidx = base + iot
            nm = []
            ni = []
            for t in range(_T):
                dx = xc - px[t]
                dy = yc - py[t]
                d2 = dx * dx + dy * dy
                m = d2 < mins[t]
                nm.append(jnp.where(m, d2, mins[t]))
                ni.append(jnp.where(m, fidx, idxs[t]))
            return tuple(nm), tuple(ni)

        mins, idxs = lax.fori_loop(0, _CHUNKS, cbody, init)

        idxv = jnp.zeros((16,), jnp.int32)
        for t in range(_T):
            gmin = jnp.min(mins[t])
            ii = jnp.where(mins[t] == gmin, idxs[t], jnp.int32(2147483647))
            gidx = jnp.min(ii)
            pstar = lax.rem(gidx, jnp.int32(20))
            gnext = jnp.where(pstar == jnp.int32(19), gidx - 1, gidx + 1)
            idxv = jnp.where(iot == t, gidx, idxv)
            idxv = jnp.where(iot == t + 8, gnext, idxv)

        gx = plsc.load_gather(xs, [idxv])
        gy = plsc.load_gather(ys, [idxv])
        outb[pl.ds(i * 32, 16)] = gx
        outb[pl.ds(i * 32 + 16, 16)] = gy
        return 0

    lax.fori_loop(0, _BPW, batch_body, 0)
    pltpu.sync_copy(outb, out_hbm.at[pl.ds(wid * (_BPW * 32), _BPW * 32)])


_sc_kernel = functools.partial(
    pl.kernel,
    out_type=jax.ShapeDtypeStruct((_B * 32,), jnp.float32),
    mesh=plsc.VectorSubcoreMesh(
        core_axis_name="c", subcore_axis_name="s", num_cores=2, num_subcores=16
    ),
    scratch_types=[
        pltpu.VMEM((_NPTS,), jnp.float32),
        pltpu.VMEM((_NPTS,), jnp.float32),
        pltpu.VMEM((112,), jnp.float32),
        pltpu.VMEM((16,), jnp.float32),
        pltpu.VMEM((_BPW * 32,), jnp.float32),
    ],
    compiler_params=pltpu.CompilerParams(needs_layout_passes=False),
)(_sc_body)


def _tc_body(ex_ref, ey_ref, sc_ref, o_ref):
    exv = ex_ref[...]
    eyv = ey_ref[...]
    s = sc_ref[...]

    pxs = [exv[:, 0:1]]
    pys = [eyv[:, 0:1]]
    for t in range(1, _T):
        pxs.append(pxs[-1] + exv[:, t:t + 1])
        pys.append(pys[-1] + eyv[:, t:t + 1])
    pxc = jnp.concatenate(pxs, axis=1)
    pyc = jnp.concatenate(pys, axis=1)

    mx = s[:, 0:6]
    nx = s[:, 8:14]
    my = s[:, 16:22]
    ny = s[:, 24:30]
    bx = nx - mx
    by = ny - my

    ax = jnp.concatenate([exv[:, 1:6], exv[:, 5:6]], axis=1)
    ay = jnp.concatenate([eyv[:, 1:6], eyv[:, 5:6]], axis=1)

    cross = ax * by - ay * bx
    dot = ax * bx + ay * by
    ac = jnp.abs(cross)
    ad = jnp.abs(dot)
    mn = jnp.minimum(ac, ad)
    mxv = jnp.maximum(ac, ad)
    q = mn / (mxv + 1e-30)
    s2 = q * q
    at = ((((-0.013887473 * s2 + 0.058559403) * s2 - 0.122270391) * s2
           + 0.196054836) * s2 - 0.333060156) * s2 + 0.999998017
    at = at * q
    yaw = jnp.where(ac <= ad, at, (math.pi / 2) - at)

    ddx = mx - pxc
    ddy = my - pyc
    dmask = (ddx * ddx + ddy * ddy) > 4.0
    sdx = pxc[:, 5:6] - pxc[:, 0:1]
    sdy = pyc[:, 5:6] - pyc[:, 0:1]
    smask = (sdx * sdx + sdy * sdy) < 1.0
    yaw = jnp.where(dmask | smask, 0.0, yaw)
    o_ref[...] = jnp.sum(yaw).reshape(1, 1) * (1.0 / (_B * _T))


_tc_call = pl.pallas_call(
    _tc_body,
    out_shape=jax.ShapeDtypeStruct((1, 1), jnp.float32),
)


def kernel(ego_fut_preds, lane_preds, lane_score_preds):
    lx = lane_preds[:, :, :, 0].reshape(_B, _NPTS)
    ly = lane_preds[:, :, :, 1].reshape(_B, _NPTS)
    sv = jnp.pad(lane_score_preds[:, :, 0], ((0, 0), (0, 12)), constant_values=1.0)
    eg = jnp.pad(ego_fut_preds.reshape(_B, 12), ((0, 0), (0, 4)))
    scout = _sc_kernel(lx, ly, sv, eg)
    ex = jnp.pad(ego_fut_preds[:, :, 0], ((0, 0), (0, 2)))
    ey = jnp.pad(ego_fut_preds[:, :, 1], ((0, 0), (0, 2)))
    out = _tc_call(ex, ey, scout.reshape(_B, 32))
    return out[0, 0]


# speedup vs baseline: 3.6732x; 1.3756x over previous
import functools
import math

import jax
import jax.numpy as jnp
from jax import lax
from jax.experimental import pallas as pl
from jax.experimental.pallas import tpu as pltpu
from jax.experimental.pallas import tpu_sc as plsc

_B = 512
_T = 6
_NPTS = 2000
_NPAD = 2048
_CH2 = _NPAD // 32
_NW = 32
_BPW = _B // _NW


def _sc_body(lx_hbm, ly_hbm, sv_hbm, eg_hbm, out_hbm,
             xsA, ysA, xsB, ysB, svA, svB, egA, egB, vtab, outb, semA, semB):
    wid = lax.axis_index("s") * 2 + lax.axis_index("c")
    iot = lax.iota(jnp.int32, 16)
    base_b = wid * _BPW

    def vinit(c, _):
        vtab[pl.ds(c * 16, 16)] = lax.div(c * 16 + iot, jnp.int32(20))
        return 0

    lax.fori_loop(0, _NPAD // 16, vinit, 0)

    def issue(b, xs, ys, sv, eg, sem):
        pltpu.async_copy(lx_hbm.at[b], xs, sem)
        pltpu.async_copy(ly_hbm.at[b], ys, sem)
        pltpu.async_copy(sv_hbm.at[b], sv, sem)
        pltpu.async_copy(eg_hbm.at[b], eg, sem)

    def wait(b, xs, ys, sv, eg, sem):
        pltpu.make_async_copy(lx_hbm.at[b], xs, sem).wait()
        pltpu.make_async_copy(ly_hbm.at[b], ys, sem).wait()
        pltpu.make_async_copy(sv_hbm.at[b], sv, sem).wait()
        pltpu.make_async_copy(eg_hbm.at[b], eg, sem).wait()

    def process(slot, xs, ys, sv, eg):
        ev = eg[...]
        px = []
        py = []
        ax_ = ev[0] * 0.0
        ay_ = ax_
        for t in range(_T):
            ax_ = ax_ + ev[2 * t]
            ay_ = ay_ + ev[2 * t + 1]
            px.append(ax_)
            py.append(ay_)

        big = jnp.full((16,), 3e38, jnp.float32)
        zi = jnp.zeros((16,), jnp.int32)
        init = (tuple(big for _ in range(_T)), tuple(zi for _ in range(_T)))

        def cbody(c, carry):
            mins, idxs = carry
            for k in range(2):
                base = c * 32 + k * 16
                xr = xs[pl.ds(base, 16)]
                yr = ys[pl.ds(base, 16)]
                vix = vtab[pl.ds(base, 16)]
                scv = plsc.load_gather(sv, [vix])
                m = scv < 0.5
                xc = jnp.where(m, 1e6, xr * 30.0 - 15.0)
                yc = jnp.where(m, 1e6, yr * 60.0 - 30.0)
                fidx = base + iot
                nm = []
                ni = []
                for t in range(_T):
                    dx = xc - px[t]
                    dy = yc - py[t]
                    d2 = dx * dx + dy * dy
                    lt = d2 < mins[t]
                    nm.append(jnp.where(lt, d2, mins[t]))
                    ni.append(jnp.where(lt, fidx, idxs[t]))
                mins = tuple(nm)
                idxs = tuple(ni)
            return mins, idxs

        mins, idxs = lax.fori_loop(0, _CH2, cbody, init)

        idxv = jnp.zeros((16,), jnp.int32)
        for t in range(_T):
            gmin = jnp.min(mins[t])
            ii = jnp.where(mins[t] == gmin, idxs[t], jnp.int32(2147483647))
            gidx = jnp.min(ii)
            pstar = lax.rem(gidx, jnp.int32(20))
            gnext = jnp.where(pstar == jnp.int32(19), gidx - 1, gidx + 1)
            idxv = jnp.where(iot == t, gidx, idxv)
            idxv = jnp.where(iot == t + 8, gnext, idxv)

        gxr = plsc.load_gather(xs, [idxv])
        gyr = plsc.load_gather(ys, [idxv])
        vig = plsc.load_gather(vtab, [idxv])
        scg = plsc.load_gather(sv, [vig])
        mg = scg < 0.5
        gx = jnp.where(mg, 1e6, gxr * 30.0 - 15.0)
        gy = jnp.where(mg, 1e6, gyr * 60.0 - 30.0)
        outb[pl.ds(slot * 32, 16)] = gx
        outb[pl.ds(slot * 32 + 16, 16)] = gy

    issue(base_b, xsA, ysA, svA, egA, semA)

    def bbody(j, _):
        b0 = base_b + 2 * j
        issue(b0 + 1, xsB, ysB, svB, egB, semB)
        wait(b0, xsA, ysA, svA, egA, semA)
        process(2 * j, xsA, ysA, svA, egA)

        @pl.when(j < _BPW // 2 - 1)
        def _():
            issue(b0 + 2, xsA, ysA, svA, egA, semA)

        wait(b0 + 1, xsB, ysB, svB, egB, semB)
        process(2 * j + 1, xsB, ysB, svB, egB)
        return 0

    lax.fori_loop(0, _BPW // 2, bbody, 0)
    pltpu.sync_copy(outb, out_hbm.at[pl.ds(wid * (_BPW * 32), _BPW * 32)])


_sc_kernel = functools.partial(
    pl.kernel,
    out_type=jax.ShapeDtypeStruct((_B * 32,), jnp.float32),
    mesh=plsc.VectorSubcoreMesh(
        core_axis_name="c", subcore_axis_name="s", num_cores=2, num_subcores=16
    ),
    scratch_types=[
        pltpu.VMEM((_NPAD,), jnp.float32),
        pltpu.VMEM((_NPAD,), jnp.float32),
        pltpu.VMEM((_NPAD,), jnp.float32),
        pltpu.VMEM((_NPAD,), jnp.float32),
        pltpu.VMEM((112,), jnp.float32),
        pltpu.VMEM((112,), jnp.float32),
        pltpu.VMEM((16,), jnp.float32),
        pltpu.VMEM((16,), jnp.float32),
        pltpu.VMEM((_NPAD,), jnp.int32),
        pltpu.VMEM((_BPW * 32,), jnp.float32),
        pltpu.SemaphoreType.DMA,
        pltpu.SemaphoreType.DMA,
    ],
    compiler_params=pltpu.CompilerParams(needs_layout_passes=False),
)(_sc_body)


def _tc_body(ex_ref, ey_ref, sc_ref, o_ref):
    exv = ex_ref[...]
    eyv = ey_ref[...]
    s = sc_ref[...]

    pxs = [exv[:, 0:1]]
    pys = [eyv[:, 0:1]]
    for t in range(1, _T):
        pxs.append(pxs[-1] + exv[:, t:t + 1])
        pys.append(pys[-1] + eyv[:, t:t + 1])
    pxc = jnp.concatenate(pxs, axis=1)
    pyc = jnp.concatenate(pys, axis=1)

    mx = s[:, 0:6]
    nx = s[:, 8:14]
    my = s[:, 16:22]
    ny = s[:, 24:30]
    bx = nx - mx
    by = ny - my

    ax = jnp.concatenate([exv[:, 1:6], exv[:, 5:6]], axis=1)
    ay = jnp.concatenate([eyv[:, 1:6], eyv[:, 5:6]], axis=1)

    cross = ax * by - ay * bx
    dot = ax * bx + ay * by
    ac = jnp.abs(cross)
    ad = jnp.abs(dot)
    mn = jnp.minimum(ac, ad)
    mxv = jnp.maximum(ac, ad)
    q = mn / (mxv + 1e-30)
    s2 = q * q
    at = ((((-0.013887473 * s2 + 0.058559403) * s2 - 0.122270391) * s2
           + 0.196054836) * s2 - 0.333060156) * s2 + 0.999998017
    at = at * q
    yaw = jnp.where(ac <= ad, at, (math.pi / 2) - at)

    ddx = mx - pxc
    ddy = my - pyc
    dmask = (ddx * ddx + ddy * ddy) > 4.0
    sdx = pxc[:, 5:6] - pxc[:, 0:1]
    sdy = pyc[:, 5:6] - pyc[:, 0:1]
    smask = (sdx * sdx + sdy * sdy) < 1.0
    yaw = jnp.where(dmask | smask, 0.0, yaw)
    o_ref[...] = jnp.sum(yaw).reshape(1, 1) * (1.0 / (_B * _T))


_tc_call = pl.pallas_call(
    _tc_body,
    out_shape=jax.ShapeDtypeStruct((1, 1), jnp.float32),
)


def kernel(ego_fut_preds, lane_preds, lane_score_preds):
    lx = jnp.pad(lane_preds[:, :, :, 0].reshape(_B, _NPTS), ((0, 0), (0, _NPAD - _NPTS)))
    ly = jnp.pad(lane_preds[:, :, :, 1].reshape(_B, _NPTS), ((0, 0), (0, _NPAD - _NPTS)))
    sv = jnp.pad(lane_score_preds[:, :, 0], ((0, 0), (0, 12)))
    eg = jnp.pad(ego_fut_preds.reshape(_B, 12), ((0, 0), (0, 4)))
    scout = _sc_kernel(lx, ly, sv, eg)
    ex = jnp.pad(ego_fut_preds[:, :, 0], ((0, 0), (0, 2)))
    ey = jnp.pad(ego_fut_preds[:, :, 1], ((0, 0), (0, 2)))
    out = _tc_call(ex, ey, scout.reshape(_B, 32))
    return out[0, 0]


# speedup vs baseline: 3.7419x; 1.0187x over previous
import functools
import math

import jax
import jax.numpy as jnp
from jax import lax
from jax.experimental import pallas as pl
from jax.experimental.pallas import tpu as pltpu
from jax.experimental.pallas import tpu_sc as plsc

_B = 512
_T = 6
_NPTS = 2000
_NPAD = 2048
_CH2 = _NPAD // 32
_NW = 32
_BPW = _B // _NW


def _sc_body(lx_hbm, ly_hbm, sv_hbm, eg_hbm, out_hbm,
             xsA, ysA, xsB, ysB, svA, svB, egA, egB, vtab, outb, semA, semB):
    wid = lax.axis_index("s") * 2 + lax.axis_index("c")
    iot = lax.iota(jnp.int32, 16)
    base_b = wid * _BPW

    def vinit(c, _):
        vtab[pl.ds(c * 16, 16)] = lax.div(c * 16 + iot, jnp.int32(20))
        return 0

    lax.fori_loop(0, _NPAD // 16, vinit, 0)

    def issue(b, xs, ys, sv, eg, sem):
        pltpu.async_copy(lx_hbm.at[b], xs, sem)
        pltpu.async_copy(ly_hbm.at[b], ys, sem)
        pltpu.async_copy(sv_hbm.at[b], sv, sem)
        pltpu.async_copy(eg_hbm.at[b], eg, sem)

    def wait(b, xs, ys, sv, eg, sem):
        pltpu.make_async_copy(lx_hbm.at[b], xs, sem).wait()
        pltpu.make_async_copy(ly_hbm.at[b], ys, sem).wait()
        pltpu.make_async_copy(sv_hbm.at[b], sv, sem).wait()
        pltpu.make_async_copy(eg_hbm.at[b], eg, sem).wait()

    def process(slot, xs, ys, sv, eg):
        ev = eg[...]
        m2p = []
        m2q = []
        ax_ = ev[0] * 0.0
        ay_ = ax_
        for t in range(_T):
            ax_ = ax_ + ev[2 * t]
            ay_ = ay_ + ev[2 * t + 1]
            m2p.append(-2.0 * ax_)
            m2q.append(-2.0 * ay_)

        big = jnp.full((16,), 3e38, jnp.float32)
        zi = jnp.zeros((16,), jnp.int32)
        init = (tuple(big for _ in range(_T)), tuple(zi for _ in range(_T)))

        def cbody(c, carry):
            mins, idxs = carry
            for k in range(2):
                base = c * 32 + k * 16
                xr = xs[pl.ds(base, 16)]
                yr = ys[pl.ds(base, 16)]
                vix = vtab[pl.ds(base, 16)]
                scv = plsc.load_gather(sv, [vix])
                m = scv < 0.5
                xc = xr * 30.0 - 15.0
                yc = yr * 60.0 - 30.0
                r2 = xc * xc + yc * yc
                r2 = jnp.where(m, 1e30, r2)
                fidx = base + iot
                nm = []
                ni = []
                for t in range(_T):
                    e = xc * m2p[t] + r2
                    e = yc * m2q[t] + e
                    lt = e < mins[t]
                    nm.append(jnp.where(lt, e, mins[t]))
                    ni.append(jnp.where(lt, fidx, idxs[t]))
                mins = tuple(nm)
                idxs = tuple(ni)
            return mins, idxs

        mins, idxs = lax.fori_loop(0, _CH2, cbody, init)

        idxv = jnp.zeros((16,), jnp.int32)
        for t in range(_T):
            gmin = jnp.min(mins[t])
            ii = jnp.where(mins[t] == gmin, idxs[t], jnp.int32(2147483647))
            gidx = jnp.min(ii)
            pstar = lax.rem(gidx, jnp.int32(20))
            gnext = jnp.where(pstar == jnp.int32(19), gidx - 1, gidx + 1)
            idxv = jnp.where(iot == t, gidx, idxv)
            idxv = jnp.where(iot == t + 8, gnext, idxv)

        gxr = plsc.load_gather(xs, [idxv])
        gyr = plsc.load_gather(ys, [idxv])
        vig = plsc.load_gather(vtab, [idxv])
        scg = plsc.load_gather(sv, [vig])
        mg = scg < 0.5
        gx = jnp.where(mg, 1e6, gxr * 30.0 - 15.0)
        gy = jnp.where(mg, 1e6, gyr * 60.0 - 30.0)
        outb[pl.ds(slot * 32, 16)] = gx
        outb[pl.ds(slot * 32 + 16, 16)] = gy

    issue(base_b, xsA, ysA, svA, egA, semA)

    def bbody(j, _):
        b0 = base_b + 2 * j
        issue(b0 + 1, xsB, ysB, svB, egB, semB)
        wait(b0, xsA, ysA, svA, egA, semA)
        process(2 * j, xsA, ysA, svA, egA)

        @pl.when(j < _BPW // 2 - 1)
        def _():
            issue(b0 + 2, xsA, ysA, svA, egA, semA)

        wait(b0 + 1, xsB, ysB, svB, egB, semB)
        process(2 * j + 1, xsB, ysB, svB, egB)
        return 0

    lax.fori_loop(0, _BPW // 2, bbody, 0)
    pltpu.sync_copy(outb, out_hbm.at[pl.ds(wid * (_BPW * 32), _BPW * 32)])


_sc_kernel = functools.partial(
    pl.kernel,
    out_type=jax.ShapeDtypeStruct((_B * 32,), jnp.float32),
    mesh=plsc.VectorSubcoreMesh(
        core_axis_name="c", subcore_axis_name="s", num_cores=2, num_subcores=16
    ),
    scratch_types=[
        pltpu.VMEM((_NPAD,), jnp.float32),
        pltpu.VMEM((_NPAD,), jnp.float32),
        pltpu.VMEM((_NPAD,), jnp.float32),
        pltpu.VMEM((_NPAD,), jnp.float32),
        pltpu.VMEM((112,), jnp.float32),
        pltpu.VMEM((112,), jnp.float32),
        pltpu.VMEM((16,), jnp.float32),
        pltpu.VMEM((16,), jnp.float32),
        pltpu.VMEM((_NPAD,), jnp.int32),
        pltpu.VMEM((_BPW * 32,), jnp.float32),
        pltpu.SemaphoreType.DMA,
        pltpu.SemaphoreType.DMA,
    ],
    compiler_params=pltpu.CompilerParams(needs_layout_passes=False),
)(_sc_body)


def _tc_body(ex_ref, ey_ref, sc_ref, o_ref):
    exv = ex_ref[...]
    eyv = ey_ref[...]
    s = sc_ref[...]

    pxs = [exv[:, 0:1]]
    pys = [eyv[:, 0:1]]
    for t in range(1, _T):
        pxs.append(pxs[-1] + exv[:, t:t + 1])
        pys.append(pys[-1] + eyv[:, t:t + 1])
    pxc = jnp.concatenate(pxs, axis=1)
    pyc = jnp.concatenate(pys, axis=1)

    mx = s[:, 0:6]
    nx = s[:, 8:14]
    my = s[:, 16:22]
    ny = s[:, 24:30]
    bx = nx - mx
    by = ny - my

    ax = jnp.concatenate([exv[:, 1:6], exv[:, 5:6]], axis=1)
    ay = jnp.concatenate([eyv[:, 1:6], eyv[:, 5:6]], axis=1)

    cross = ax * by - ay * bx
    dot = ax * bx + ay * by
    ac = jnp.abs(cross)
    ad = jnp.abs(dot)
    mn = jnp.minimum(ac, ad)
    mxv = jnp.maximum(ac, ad)
    q = mn / (mxv + 1e-30)
    s2 = q * q
    at = ((((-0.013887473 * s2 + 0.058559403) * s2 - 0.122270391) * s2
           + 0.196054836) * s2 - 0.333060156) * s2 + 0.999998017
    at = at * q
    yaw = jnp.where(ac <= ad, at, (math.pi / 2) - at)

    ddx = mx - pxc
    ddy = my - pyc
    dmask = (ddx * ddx + ddy * ddy) > 4.0
    sdx = pxc[:, 5:6] - pxc[:, 0:1]
    sdy = pyc[:, 5:6] - pyc[:, 0:1]
    smask = (sdx * sdx + sdy * sdy) < 1.0
    yaw = jnp.where(dmask | smask, 0.0, yaw)
    o_ref[...] = jnp.sum(yaw).reshape(1, 1) * (1.0 / (_B * _T))


_tc_call = pl.pallas_call(
    _tc_body,
    out_shape=jax.ShapeDtypeStruct((1, 1), jnp.float32),
)


def kernel(ego_fut_preds, lane_preds, lane_score_preds):
    lx = jnp.pad(lane_preds[:, :, :, 0].reshape(_B, _NPTS), ((0, 0), (0, _NPAD - _NPTS)))
    ly = jnp.pad(lane_preds[:, :, :, 1].reshape(_B, _NPTS), ((0, 0), (0, _NPAD - _NPTS)))
    sv = jnp.pad(lane_score_preds[:, :, 0], ((0, 0), (0, 12)))
    eg = jnp.pad(ego_fut_preds.reshape(_B, 12), ((0, 0), (0, 4)))
    scout = _sc_kernel(lx, ly, sv, eg)
    ex = jnp.pad(ego_fut_preds[:, :, 0], ((0, 0), (0, 2)))
    ey = jnp.pad(ego_fut_preds[:, :, 1], ((0, 0), (0, 2)))
    out = _tc_call(ex, ey, scout.reshape(_B, 32))
    return out[0, 0]


# speedup vs baseline: 5.4459x; 1.4554x over previous
import functools
import math

import jax
import jax.numpy as jnp
from jax import lax
from jax.experimental import pallas as pl
from jax.experimental.pallas import tpu as pltpu
from jax.experimental.pallas import tpu_sc as plsc

_B = 512
_T = 6
_NPTS = 2000
_NPAD = 2048
_CH2 = _NPAD // 32
_NW = 32
_BPW = _B // _NW


def _sc_body(lxy_hbm, sg_hbm, out_hbm, bufA, bufB, sgA, sgB, vtab, outb, semA, semB):
    wid = lax.axis_index("s") * 2 + lax.axis_index("c")
    iot = lax.iota(jnp.int32, 16)
    base_b = wid * _BPW

    def vinit(c, _):
        vtab[pl.ds(c * 16, 16)] = lax.div(c * 16 + iot, jnp.int32(20))
        return 0

    lax.fori_loop(0, _NPAD // 16, vinit, 0)

    def issue(b, buf, sg, sem):
        pltpu.async_copy(lxy_hbm.at[b], buf, sem)
        pltpu.async_copy(sg_hbm.at[b], sg, sem)

    def wait(b, buf, sg, sem):
        pltpu.make_async_copy(lxy_hbm.at[b], buf, sem).wait()
        pltpu.make_async_copy(sg_hbm.at[b], sg, sem).wait()

    def process(slot, buf, sg):
        ev = sg[pl.ds(112, 16)]
        m2p = []
        m2q = []
        ax_ = ev[0] * 0.0
        ay_ = ax_
        for t in range(_T):
            ax_ = ax_ + ev[2 * t]
            ay_ = ay_ + ev[2 * t + 1]
            m2p.append(-2.0 * ax_)
            m2q.append(-2.0 * ay_)

        big = jnp.full((16,), 3e38, jnp.float32)
        zi = jnp.zeros((16,), jnp.int32)
        init = (tuple(big for _ in range(_T)), tuple(zi for _ in range(_T)))
        iot2 = iot * 2

        def cbody(c, carry):
            mins, idxs = carry
            for k in range(2):
                base = c * 32 + k * 16
                off = jnp.minimum(base * 2 + iot2, jnp.int32(3998))
                xr = plsc.load_gather(buf, [off])
                yr = plsc.load_gather(buf, [off + 1])
                vix = vtab[pl.ds(base, 16)]
                scv = plsc.load_gather(sg, [vix])
                m = scv < 0.5
                xc = xr * 30.0 - 15.0
                yc = yr * 60.0 - 30.0
                r2 = xc * xc + yc * yc
                r2 = jnp.where(m, 1e30, r2)
                fidx = base + iot
                nm = []
                ni = []
                for t in range(_T):
                    e = xc * m2p[t] + r2
                    e = yc * m2q[t] + e
                    lt = e < mins[t]
                    nm.append(jnp.where(lt, e, mins[t]))
                    ni.append(jnp.where(lt, fidx, idxs[t]))
                mins = tuple(nm)
                idxs = tuple(ni)
            return mins, idxs

        mins, idxs = lax.fori_loop(0, _CH2, cbody, init)

        idxv = jnp.zeros((16,), jnp.int32)
        for t in range(_T):
            gmin = jnp.min(mins[t])
            ii = jnp.where(mins[t] == gmin, idxs[t], jnp.int32(2147483647))
            gidx = jnp.min(ii)
            pstar = lax.rem(gidx, jnp.int32(20))
            gnext = jnp.where(pstar == jnp.int32(19), gidx - 1, gidx + 1)
            idxv = jnp.where(iot == t, gidx, idxv)
            idxv = jnp.where(iot == t + 8, gnext, idxv)

        gxr = plsc.load_gather(buf, [idxv * 2])
        gyr = plsc.load_gather(buf, [idxv * 2 + 1])
        vig = plsc.load_gather(vtab, [idxv])
        scg = plsc.load_gather(sg, [vig])
        mg = scg < 0.5
        gx = jnp.where(mg, 1e6, gxr * 30.0 - 15.0)
        gy = jnp.where(mg, 1e6, gyr * 60.0 - 30.0)
        outb[pl.ds(slot * 32, 16)] = gx
        outb[pl.ds(slot * 32 + 16, 16)] = gy

    issue(base_b, bufA, sgA, semA)

    def bbody(j, _):
        b0 = base_b + 2 * j
        issue(b0 + 1, bufB, sgB, semB)
        wait(b0, bufA, sgA, semA)
        process(2 * j, bufA, sgA)

        @pl.when(j < _BPW // 2 - 1)
        def _():
            issue(b0 + 2, bufA, sgA, semA)

        wait(b0 + 1, bufB, sgB, semB)
        process(2 * j + 1, bufB, sgB)
        return 0

    lax.fori_loop(0, _BPW // 2, bbody, 0)
    pltpu.sync_copy(outb, out_hbm.at[pl.ds(wid * (_BPW * 32), _BPW * 32)])


_sc_kernel = functools.partial(
    pl.kernel,
    out_type=jax.ShapeDtypeStruct((_B * 32,), jnp.float32),
    mesh=plsc.VectorSubcoreMesh(
        core_axis_name="c", subcore_axis_name="s", num_cores=2, num_subcores=16
    ),
    scratch_types=[
        pltpu.VMEM((2 * _NPTS,), jnp.float32),
        pltpu.VMEM((2 * _NPTS,), jnp.float32),
        pltpu.VMEM((128,), jnp.float32),
        pltpu.VMEM((128,), jnp.float32),
        pltpu.VMEM((_NPAD,), jnp.int32),
        pltpu.VMEM((_BPW * 32,), jnp.float32),
        pltpu.SemaphoreType.DMA,
        pltpu.SemaphoreType.DMA,
    ],
    compiler_params=pltpu.CompilerParams(needs_layout_passes=False),
)(_sc_body)


def _tc_body(ex_ref, ey_ref, sc_ref, o_ref):
    exv = ex_ref[...]
    eyv = ey_ref[...]
    s = sc_ref[...]

    pxs = [exv[:, 0:1]]
    pys = [eyv[:, 0:1]]
    for t in range(1, _T):
        pxs.append(pxs[-1] + exv[:, t:t + 1])
        pys.append(pys[-1] + eyv[:, t:t + 1])
    pxc = jnp.concatenate(pxs, axis=1)
    pyc = jnp.concatenate(pys, axis=1)

    mx = s[:, 0:6]
    nx = s[:, 8:14]
    my = s[:, 16:22]
    ny = s[:, 24:30]
    bx = nx - mx
    by = ny - my

    ax = jnp.concatenate([exv[:, 1:6], exv[:, 5:6]], axis=1)
    ay = jnp.concatenate([eyv[:, 1:6], eyv[:, 5:6]], axis=1)

    cross = ax * by - ay * bx
    dot = ax * bx + ay * by
    ac = jnp.abs(cross)
    ad = jnp.abs(dot)
    mn = jnp.minimum(ac, ad)
    mxv = jnp.maximum(ac, ad)
    q = mn / (mxv + 1e-30)
    s2 = q * q
    at = ((((-0.013887473 * s2 + 0.058559403) * s2 - 0.122270391) * s2
           + 0.196054836) * s2 - 0.333060156) * s2 + 0.999998017
    at = at * q
    yaw = jnp.where(ac <= ad, at, (math.pi / 2) - at)

    ddx = mx - pxc
    ddy = my - pyc
    dmask = (ddx * ddx + ddy * ddy) > 4.0
    sdx = pxc[:, 5:6] - pxc[:, 0:1]
    sdy = pyc[:, 5:6] - pyc[:, 0:1]
    smask = (sdx * sdx + sdy * sdy) < 1.0
    yaw = jnp.where(dmask | smask, 0.0, yaw)
    o_ref[...] = jnp.sum(yaw).reshape(1, 1) * (1.0 / (_B * _T))


_tc_call = pl.pallas_call(
    _tc_body,
    out_shape=jax.ShapeDtypeStruct((1, 1), jnp.float32),
)


def kernel(ego_fut_preds, lane_preds, lane_score_preds):
    lxy = lane_preds.reshape(_B, 2 * _NPTS)
    eg = ego_fut_preds.reshape(_B, 12)
    z12 = jnp.zeros((_B, 12), jnp.float32)
    z4 = jnp.zeros((_B, 4), jnp.float32)
    sg = jnp.concatenate([lane_score_preds[:, :, 0], z12, eg, z4], axis=1)
    scout = _sc_kernel(lxy, sg)
    ex = jnp.pad(ego_fut_preds[:, :, 0], ((0, 0), (0, 2)))
    ey = jnp.pad(ego_fut_preds[:, :, 1], ((0, 0), (0, 2)))
    out = _tc_call(ex, ey, scout.reshape(_B, 32))
    return out[0, 0]
